# Initial kernel scaffold; baseline (speedup 1.0000x reference)
import jax
import jax.numpy as jnp
from jax.experimental import pallas as pl


def kernel(x, edge_attr, pos, edge_index, eW1, eb1, eW2, eb2, eg, ebt, nW1, nb1, nW2, nb2, ng, nbt):
    raise NotImplementedError("write your pallas kernel here")


# speedup vs baseline: 4.1194x; 4.1194x over previous
import functools

import jax
import jax.numpy as jnp
from jax import lax
from jax.experimental import pallas as pl
from jax.experimental.pallas import tpu as pltpu
from jax.experimental.pallas import tpu_sc as plsc

N = 10000
E = 160000
D = 128
NB = 4

NC = 2
NS = 16
NW = NC * NS
CHUNK = 128
NCHUNK = E // CHUNK
FULL_T = NCHUNK // NW
N_PAD = 10112
ROWS_PER_SUB = N_PAD // NS


def _sc_segment_sum(edge3d, recv, zeros):
    mesh = plsc.VectorSubcoreMesh(core_axis_name="c", subcore_axis_name="s")

    @functools.partial(
        pl.kernel,
        out_type=jax.ShapeDtypeStruct((NC * N_PAD, D), jnp.float32),
        mesh=mesh,
        scratch_types=[
            pltpu.VMEM((1, CHUNK), jnp.int32),
            pltpu.VMEM((CHUNK, D), jnp.float32),
            pltpu.VMEM_SHARED((N_PAD, D), jnp.float32),
        ],
    )
    def seg(edge_hbm, recv_hbm, zeros_hbm, out_hbm, idx_v, rows_v, acc):
        c = lax.axis_index("c")
        s = lax.axis_index("s")
        w = c * NS + s
        rbase = s * ROWS_PER_SUB
        pltpu.sync_copy(zeros_hbm.at[pl.ds(rbase, ROWS_PER_SUB)],
                        acc.at[pl.ds(rbase, ROWS_PER_SUB)])
        plsc.subcore_barrier()

        def one_chunk(ch):
            pltpu.sync_copy(recv_hbm.at[pl.ds(ch * CHUNK, CHUNK)], idx_v.at[0])
            pltpu.sync_copy(edge_hbm.at[ch], rows_v)
            pltpu.sync_copy(rows_v, acc.at[idx_v.at[0]], add=True)

        def body(t, carry):
            one_chunk(t * NW + w)
            return carry

        lax.fori_loop(0, FULL_T, body, 0)
        @pl.when(w < NCHUNK - FULL_T * NW)
        def _():
            one_chunk(FULL_T * NW + w)

        plsc.subcore_barrier()
        pltpu.sync_copy(acc.at[pl.ds(rbase, ROWS_PER_SUB)],
                        out_hbm.at[pl.ds(c * N_PAD + rbase, ROWS_PER_SUB)])

    return seg(edge3d, recv, zeros)


BN = 1000


def _tc_body(x_ref, p0_ref, p1_ref, w1a_ref, w1b_ref, b1_ref, w2_ref, b2_ref,
             g_ref, bt_ref, o_ref):
    xn = x_ref[...]
    agg = p0_ref[...] + p1_ref[...]
    for i in range(NB):
        u = (jnp.dot(xn, w1a_ref[i], preferred_element_type=jnp.float32)
             + jnp.dot(agg, w1b_ref[i], preferred_element_type=jnp.float32)
             + b1_ref[i][None, :])
        h = jnp.maximum(u, 0.0)
        v = jnp.dot(h, w2_ref[i], preferred_element_type=jnp.float32) + b2_ref[i][None, :]
        mu = jnp.mean(v, axis=-1, keepdims=True)
        var = jnp.mean((v - mu) ** 2, axis=-1, keepdims=True)
        ln = (v - mu) / jnp.sqrt(var + 1e-5) * g_ref[i][None, :] + bt_ref[i][None, :]
        xn = ln + xn
    o_ref[...] = xn


def _tc_mlp(x, p0, p1, w1a, w1b, nb1, nW2, nb2, ng, nbt):
    whole = lambda shape: pl.BlockSpec(shape, lambda i: (0,) * len(shape))
    return pl.pallas_call(
        _tc_body,
        grid=(N // BN,),
        in_specs=[
            pl.BlockSpec((BN, D), lambda i: (i, 0)),
            pl.BlockSpec((BN, D), lambda i: (i, 0)),
            pl.BlockSpec((BN, D), lambda i: (i, 0)),
            whole((NB, D, D)),
            whole((NB, D, D)),
            whole((NB, D)),
            whole((NB, D, D)),
            whole((NB, D)),
            whole((NB, D)),
            whole((NB, D)),
        ],
        out_specs=pl.BlockSpec((BN, D), lambda i: (i, 0)),
        out_shape=jax.ShapeDtypeStruct((N, D), jnp.float32),
    )(x, p0, p1, w1a, w1b, nb1, nW2, nb2, ng, nbt)


def kernel(x, edge_attr, pos, edge_index, eW1, eb1, eW2, eb2, eg, ebt,
           nW1, nb1, nW2, nb2, ng, nbt):
    edge3d = edge_attr.reshape(NCHUNK, CHUNK, D)
    recv = edge_index[1]
    zeros = jnp.zeros((N_PAD, D), jnp.float32)
    partials = _sc_segment_sum(edge3d, recv, zeros)
    p0 = partials[:N]
    p1 = partials[N_PAD:N_PAD + N]
    w1a = nW1[:, :D, :]
    w1b = nW1[:, D:, :]
    return _tc_mlp(x, p0, p1, w1a, w1b, nb1, nW2, nb2, ng, nbt)


# speedup vs baseline: 5.0424x; 1.2241x over previous
import functools

import jax
import jax.numpy as jnp
from jax import lax
from jax.experimental import pallas as pl
from jax.experimental.pallas import tpu as pltpu
from jax.experimental.pallas import tpu_sc as plsc

N = 10000
E = 160000
D = 128
NB = 4

NC = 2
NS = 16
NW = NC * NS
CHUNK = 128
NCHUNK = E // CHUNK
FULL_T = NCHUNK // NW
BN = 400
N_PAD = 10112
ROWS_PER_SUB = N_PAD // NS
P1OFF = 10400


def _sc_segment_sum(edge_attr, recv):
    mesh = plsc.VectorSubcoreMesh(core_axis_name="c", subcore_axis_name="s")

    @functools.partial(
        pl.kernel,
        out_type=jax.ShapeDtypeStruct((P1OFF + N_PAD, D), jnp.float32),
        mesh=mesh,
        scratch_types=[
            pltpu.VMEM((2, CHUNK), jnp.int32),
            pltpu.VMEM((2, CHUNK, D), jnp.float32),
            pltpu.VMEM((CHUNK, D), jnp.float32),
            pltpu.VMEM_SHARED((N_PAD, D), jnp.float32),
            pltpu.SemaphoreType.DMA,
            pltpu.SemaphoreType.DMA,
        ],
    )
    def seg(edge_hbm, recv_hbm, out_hbm, idx_v, rows_v, zbuf, acc, sem0, sem1):
        c = lax.axis_index("c")
        s = lax.axis_index("s")
        w = c * NS + s
        rbase = s * ROWS_PER_SUB
        sems = (sem0, sem1)

        def zrow(r, carry):
            for k in range(D // 16):
                zbuf[r, pl.ds(k * 16, 16)] = jnp.zeros((16,), jnp.float32)
            return carry

        lax.fori_loop(0, CHUNK, zrow, 0)
        for k in range(ROWS_PER_SUB // CHUNK):
            pltpu.sync_copy(zbuf, acc.at[pl.ds(rbase + k * CHUNK, CHUNK)])
        rem = ROWS_PER_SUB % CHUNK
        if rem:
            pltpu.sync_copy(zbuf.at[pl.ds(0, rem)],
                            acc.at[pl.ds(rbase + ROWS_PER_SUB - rem, rem)])
        plsc.subcore_barrier()

        def start(t, b):
            ch = t * NW + w
            pltpu.async_copy(recv_hbm.at[pl.ds(ch * CHUNK, CHUNK)],
                             idx_v.at[b], sems[b])
            pltpu.async_copy(edge_hbm.at[pl.ds(ch * CHUNK, CHUNK)],
                             rows_v.at[b], sems[b])

        def wait_fetch(b):
            pltpu.make_async_copy(recv_hbm.at[pl.ds(0, CHUNK)],
                                  idx_v.at[b], sems[b]).wait()
            pltpu.make_async_copy(edge_hbm.at[pl.ds(0, CHUNK)],
                                  rows_v.at[b], sems[b]).wait()

        def consume(b):
            pltpu.sync_copy(rows_v.at[b], acc.at[idx_v.at[b]], add=True)

        start(0, 0)
        start(1, 1)

        def ring(g2, carry):
            for b in range(2):
                t = 2 * g2 + b
                wait_fetch(b)
                consume(b)

                @pl.when(t + 2 < FULL_T)
                def _():
                    start(t + 2, b)
            return carry

        lax.fori_loop(0, (FULL_T - 1) // 2, ring, 0)
        wait_fetch(0)
        consume(0)

        @pl.when(w < NCHUNK - FULL_T * NW)
        def _():
            ch = FULL_T * NW + w
            pltpu.sync_copy(recv_hbm.at[pl.ds(ch * CHUNK, CHUNK)], idx_v.at[1])
            pltpu.sync_copy(edge_hbm.at[pl.ds(ch * CHUNK, CHUNK)], rows_v.at[1])
            consume(1)

        plsc.subcore_barrier()
        pltpu.sync_copy(acc.at[pl.ds(rbase, ROWS_PER_SUB)],
                        out_hbm.at[pl.ds(c * P1OFF + rbase, ROWS_PER_SUB)])

    return seg(edge_attr, recv)


def _tc_body(x_ref, p0_ref, p1_ref, w1a_ref, w1b_ref, b1_ref, w2_ref, b2_ref,
             g_ref, bt_ref, o_ref):
    xn = x_ref[...]
    agg = p0_ref[...] + p1_ref[...]
    for i in range(NB):
        u = (jnp.dot(xn, w1a_ref[i], preferred_element_type=jnp.float32)
             + jnp.dot(agg, w1b_ref[i], preferred_element_type=jnp.float32)
             + b1_ref[i][None, :])
        h = jnp.maximum(u, 0.0)
        v = jnp.dot(h, w2_ref[i], preferred_element_type=jnp.float32) + b2_ref[i][None, :]
        mu = jnp.mean(v, axis=-1, keepdims=True)
        var = jnp.mean((v - mu) ** 2, axis=-1, keepdims=True)
        ln = (v - mu) / jnp.sqrt(var + 1e-5) * g_ref[i][None, :] + bt_ref[i][None, :]
        xn = ln + xn
    o_ref[...] = xn


def _tc_mlp(x, partials, w1a, w1b, nb1, nW2, nb2, ng, nbt):
    whole = lambda shape: pl.BlockSpec(shape, lambda i: (0,) * len(shape))
    return pl.pallas_call(
        _tc_body,
        grid=(N // BN,),
        in_specs=[
            pl.BlockSpec((BN, D), lambda i: (i, 0)),
            pl.BlockSpec((BN, D), lambda i: (i, 0)),
            pl.BlockSpec((BN, D), lambda i: (i + P1OFF // BN, 0)),
            whole((NB, D, D)),
            whole((NB, D, D)),
            whole((NB, D)),
            whole((NB, D, D)),
            whole((NB, D)),
            whole((NB, D)),
            whole((NB, D)),
        ],
        out_specs=pl.BlockSpec((BN, D), lambda i: (i, 0)),
        out_shape=jax.ShapeDtypeStruct((N, D), jnp.float32),
    )(x, partials, partials, w1a, w1b, nb1, nW2, nb2, ng, nbt)


def kernel(x, edge_attr, pos, edge_index, eW1, eb1, eW2, eb2, eg, ebt,
           nW1, nb1, nW2, nb2, ng, nbt):
    recv = edge_index[1]
    partials = _sc_segment_sum(edge_attr, recv)
    w1a = nW1[:, :D, :]
    w1b = nW1[:, D:, :]
    return _tc_mlp(x, partials, w1a, w1b, nb1, nW2, nb2, ng, nbt)


# speedup vs baseline: 6.3780x; 1.2649x over previous
import functools

import jax
import jax.numpy as jnp
from jax import lax
from jax.experimental import pallas as pl
from jax.experimental.pallas import tpu as pltpu
from jax.experimental.pallas import tpu_sc as plsc

N = 10000
E = 160000
D = 128
NB = 4

NC = 2
NS = 16
NW = NC * NS
CHUNK = 128
NCHUNK = E // CHUNK
FULL_T = NCHUNK // NW
BN = 1000
N_PAD = 10112
ROWS_PER_SUB = N_PAD // NS
P1OFF = 11000


def _sc_segment_sum(edge_attr, edge_index):
    mesh = plsc.VectorSubcoreMesh(core_axis_name="c", subcore_axis_name="s")

    @functools.partial(
        pl.kernel,
        out_type=jax.ShapeDtypeStruct((P1OFF + N_PAD, D), jnp.float32),
        mesh=mesh,
        scratch_types=[
            pltpu.VMEM((2, 2, CHUNK), jnp.int32),
            pltpu.VMEM((2, CHUNK, D), jnp.float32),
            pltpu.VMEM((CHUNK, D), jnp.float32),
            pltpu.VMEM_SHARED((N_PAD, D), jnp.float32),
            pltpu.SemaphoreType.DMA,
            pltpu.SemaphoreType.DMA,
        ],
    )
    def seg(edge_hbm, ei_hbm, out_hbm, idx_v, rows_v, zbuf, acc, sem0, sem1):
        c = lax.axis_index("c")
        s = lax.axis_index("s")
        w = c * NS + s
        rbase = s * ROWS_PER_SUB
        sems = (sem0, sem1)

        def zrow(r, carry):
            for k in range(D // 16):
                zbuf[r, pl.ds(k * 16, 16)] = jnp.zeros((16,), jnp.float32)
            return carry

        lax.fori_loop(0, CHUNK, zrow, 0)
        for k in range(ROWS_PER_SUB // CHUNK):
            pltpu.sync_copy(zbuf, acc.at[pl.ds(rbase + k * CHUNK, CHUNK)])
        rem = ROWS_PER_SUB % CHUNK
        if rem:
            pltpu.sync_copy(zbuf.at[pl.ds(0, rem)],
                            acc.at[pl.ds(rbase + ROWS_PER_SUB - rem, rem)])
        plsc.subcore_barrier()

        def start(t, b):
            ch = t * NW + w
            pltpu.async_copy(ei_hbm.at[:, pl.ds(ch * CHUNK, CHUNK)],
                             idx_v.at[b], sems[b])
            pltpu.async_copy(edge_hbm.at[pl.ds(ch * CHUNK, CHUNK)],
                             rows_v.at[b], sems[b])

        def wait_fetch(b):
            pltpu.make_async_copy(ei_hbm.at[:, pl.ds(0, CHUNK)],
                                  idx_v.at[b], sems[b]).wait()
            pltpu.make_async_copy(edge_hbm.at[pl.ds(0, CHUNK)],
                                  rows_v.at[b], sems[b]).wait()

        def consume(b):
            pltpu.sync_copy(rows_v.at[b], acc.at[idx_v.at[b, 1]], add=True)

        start(0, 0)
        start(1, 1)

        def ring(g2, carry):
            for b in range(2):
                t = 2 * g2 + b
                wait_fetch(b)
                consume(b)

                @pl.when(t + 2 < FULL_T)
                def _():
                    start(t + 2, b)
            return carry

        lax.fori_loop(0, (FULL_T - 1) // 2, ring, 0)
        wait_fetch(0)
        consume(0)

        @pl.when(w < NCHUNK - FULL_T * NW)
        def _():
            ch = FULL_T * NW + w
            pltpu.sync_copy(ei_hbm.at[:, pl.ds(ch * CHUNK, CHUNK)], idx_v.at[1])
            pltpu.sync_copy(edge_hbm.at[pl.ds(ch * CHUNK, CHUNK)], rows_v.at[1])
            consume(1)

        plsc.subcore_barrier()
        pltpu.sync_copy(acc.at[pl.ds(rbase, ROWS_PER_SUB)],
                        out_hbm.at[pl.ds(c * P1OFF + rbase, ROWS_PER_SUB)])

    return seg(edge_attr, edge_index)


def _tc_body(x_ref, p0_ref, p1_ref, w1a_ref, w1bcat_ref, b1_ref, w2_ref,
             b2_ref, g_ref, bt_ref, o_ref):
    xn = x_ref[...]
    agg = p0_ref[...] + p1_ref[...]
    agg4 = jnp.dot(agg.astype(jnp.bfloat16), w1bcat_ref[...],
                   preferred_element_type=jnp.float32)
    for i in range(NB):
        u = (jnp.dot(xn.astype(jnp.bfloat16), w1a_ref[i],
                     preferred_element_type=jnp.float32)
             + agg4[:, i * D:(i + 1) * D]
             + b1_ref[i][None, :])
        h = jnp.maximum(u, 0.0)
        v = (jnp.dot(h.astype(jnp.bfloat16), w2_ref[i],
                     preferred_element_type=jnp.float32)
             + b2_ref[i][None, :])
        mu = jnp.mean(v, axis=-1, keepdims=True)
        var = jnp.mean((v - mu) ** 2, axis=-1, keepdims=True)
        ln = (v - mu) / jnp.sqrt(var + 1e-5) * g_ref[i][None, :] + bt_ref[i][None, :]
        xn = ln + xn
    o_ref[...] = xn


def _tc_mlp(x, partials, w1a, w1bcat, nb1, nW2, nb2, ng, nbt):
    whole = lambda shape: pl.BlockSpec(shape, lambda i: (0,) * len(shape))
    return pl.pallas_call(
        _tc_body,
        grid=(N // BN,),
        in_specs=[
            pl.BlockSpec((BN, D), lambda i: (i, 0)),
            pl.BlockSpec((BN, D), lambda i: (i, 0)),
            pl.BlockSpec((BN, D), lambda i: (i + P1OFF // BN, 0)),
            whole((NB, D, D)),
            whole((D, NB * D)),
            whole((NB, D)),
            whole((NB, D, D)),
            whole((NB, D)),
            whole((NB, D)),
            whole((NB, D)),
        ],
        out_specs=pl.BlockSpec((BN, D), lambda i: (i, 0)),
        out_shape=jax.ShapeDtypeStruct((N, D), jnp.float32),
    )(x, partials, partials, w1a, w1bcat, nb1, nW2, nb2, ng, nbt)


def kernel(x, edge_attr, pos, edge_index, eW1, eb1, eW2, eb2, eg, ebt,
           nW1, nb1, nW2, nb2, ng, nbt):
    partials = _sc_segment_sum(edge_attr, edge_index)
    w1a = nW1[:, :D, :].astype(jnp.bfloat16)
    w1bcat = nW1[:, D:, :].transpose(1, 0, 2).reshape(D, NB * D).astype(jnp.bfloat16)
    w2 = nW2.astype(jnp.bfloat16)
    return _tc_mlp(x, partials, w1a, w1bcat, nb1, w2, nb2, ng, nbt)


# speedup vs baseline: 7.4255x; 1.1642x over previous
import functools

import jax
import jax.numpy as jnp
from jax import lax
from jax.experimental import pallas as pl
from jax.experimental.pallas import tpu as pltpu
from jax.experimental.pallas import tpu_sc as plsc

N = 10000
E = 160000
D = 128
NB = 4

NC = 2
NS = 16
NW = NC * NS
CHUNK = 128
NCHUNK = E // CHUNK
FULL_T = NCHUNK // NW
BN = 2000
N_PAD = 10112
ROWS_PER_SUB = N_PAD // NS
P1OFF = 12000


def _sc_segment_sum(edge_attr, edge_index):
    mesh = plsc.VectorSubcoreMesh(core_axis_name="c", subcore_axis_name="s")

    @functools.partial(
        pl.kernel,
        out_type=jax.ShapeDtypeStruct((P1OFF + N_PAD, D), jnp.float32),
        mesh=mesh,
        scratch_types=[
            pltpu.VMEM((2, 2, CHUNK), jnp.int32),
            pltpu.VMEM((2, CHUNK, D), jnp.float32),
            pltpu.VMEM((CHUNK, D), jnp.float32),
            pltpu.VMEM_SHARED((N_PAD, D), jnp.float32),
            pltpu.SemaphoreType.DMA,
            pltpu.SemaphoreType.DMA,
        ],
    )
    def seg(edge_hbm, ei_hbm, out_hbm, idx_v, rows_v, zbuf, acc, sem0, sem1):
        c = lax.axis_index("c")
        s = lax.axis_index("s")
        w = c * NS + s
        rbase = s * ROWS_PER_SUB
        sems = (sem0, sem1)

        def start(t, b):
            ch = t * NW + w
            pltpu.async_copy(ei_hbm.at[:, pl.ds(ch * CHUNK, CHUNK)],
                             idx_v.at[b], sems[b])
            pltpu.async_copy(edge_hbm.at[pl.ds(ch * CHUNK, CHUNK)],
                             rows_v.at[b], sems[b])

        start(0, 0)
        start(1, 1)

        def zrow(r, carry):
            for k in range(D // 16):
                zbuf[r, pl.ds(k * 16, 16)] = jnp.zeros((16,), jnp.float32)
            return carry

        lax.fori_loop(0, CHUNK, zrow, 0)
        for k in range(ROWS_PER_SUB // CHUNK):
            pltpu.sync_copy(zbuf, acc.at[pl.ds(rbase + k * CHUNK, CHUNK)])
        rem = ROWS_PER_SUB % CHUNK
        if rem:
            pltpu.sync_copy(zbuf.at[pl.ds(0, rem)],
                            acc.at[pl.ds(rbase + ROWS_PER_SUB - rem, rem)])
        plsc.subcore_barrier()

        def wait_fetch(b):
            pltpu.make_async_copy(ei_hbm.at[:, pl.ds(0, CHUNK)],
                                  idx_v.at[b], sems[b]).wait()
            pltpu.make_async_copy(edge_hbm.at[pl.ds(0, CHUNK)],
                                  rows_v.at[b], sems[b]).wait()

        def consume(b):
            pltpu.sync_copy(rows_v.at[b], acc.at[idx_v.at[b, 1]], add=True)

        def ring(g2, carry):
            for b in range(2):
                t = 2 * g2 + b
                wait_fetch(b)
                consume(b)

                @pl.when(t + 2 < FULL_T)
                def _():
                    start(t + 2, b)
            return carry

        lax.fori_loop(0, (FULL_T - 1) // 2, ring, 0)
        wait_fetch(0)
        consume(0)

        @pl.when(w < NCHUNK - FULL_T * NW)
        def _():
            ch = FULL_T * NW + w
            pltpu.sync_copy(ei_hbm.at[:, pl.ds(ch * CHUNK, CHUNK)], idx_v.at[1])
            pltpu.sync_copy(edge_hbm.at[pl.ds(ch * CHUNK, CHUNK)], rows_v.at[1])
            consume(1)

        plsc.subcore_barrier()
        pltpu.sync_copy(acc.at[pl.ds(rbase, ROWS_PER_SUB)],
                        out_hbm.at[pl.ds(c * P1OFF + rbase, ROWS_PER_SUB)])

    return seg(edge_attr, edge_index)


def _tc_body(x_ref, p0_ref, p1_ref, w1a_ref, w1bcat_ref, b1_ref, w2_ref,
             b2_ref, g_ref, bt_ref, o_ref):
    xn = x_ref[...]
    agg = p0_ref[...] + p1_ref[...]
    agg4 = (jnp.dot(agg.astype(jnp.bfloat16), w1bcat_ref[...],
                    preferred_element_type=jnp.float32)
            + b1_ref[...][None, :])
    for i in range(NB):
        u = (jnp.dot(xn.astype(jnp.bfloat16), w1a_ref[i],
                     preferred_element_type=jnp.float32)
             + agg4[:, i * D:(i + 1) * D])
        h = jnp.maximum(u, 0.0)
        v = (jnp.dot(h.astype(jnp.bfloat16), w2_ref[i],
                     preferred_element_type=jnp.float32)
             + b2_ref[i][None, :])
        mu = jnp.mean(v, axis=-1, keepdims=True)
        var = jnp.mean(v * v, axis=-1, keepdims=True) - mu * mu
        scale = lax.rsqrt(var + 1e-5) * g_ref[i][None, :]
        xn = (v - mu) * scale + bt_ref[i][None, :] + xn
    o_ref[...] = xn


def _tc_mlp(x, partials, w1a, w1bcat, nb1, nW2, nb2, ng, nbt):
    whole = lambda shape: pl.BlockSpec(shape, lambda i: (0,) * len(shape))
    return pl.pallas_call(
        _tc_body,
        grid=(N // BN,),
        in_specs=[
            pl.BlockSpec((BN, D), lambda i: (i, 0)),
            pl.BlockSpec((BN, D), lambda i: (i, 0)),
            pl.BlockSpec((BN, D), lambda i: (i + P1OFF // BN, 0)),
            whole((NB, D, D)),
            whole((D, NB * D)),
            whole((NB * D,)),
            whole((NB, D, D)),
            whole((NB, D)),
            whole((NB, D)),
            whole((NB, D)),
        ],
        out_specs=pl.BlockSpec((BN, D), lambda i: (i, 0)),
        out_shape=jax.ShapeDtypeStruct((N, D), jnp.float32),
    )(x, partials, partials, w1a, w1bcat, nb1, nW2, nb2, ng, nbt)


def kernel(x, edge_attr, pos, edge_index, eW1, eb1, eW2, eb2, eg, ebt,
           nW1, nb1, nW2, nb2, ng, nbt):
    partials = _sc_segment_sum(edge_attr, edge_index)
    w1a = nW1[:, :D, :].astype(jnp.bfloat16)
    w1bcat = nW1[:, D:, :].transpose(1, 0, 2).reshape(D, NB * D).astype(jnp.bfloat16)
    w2 = nW2.astype(jnp.bfloat16)
    b1cat = nb1.reshape(NB * D)
    return _tc_mlp(x, partials, w1a, w1bcat, b1cat, w2, nb2, ng, nbt)
